# Initial kernel scaffold; baseline (speedup 1.0000x reference)
import jax
import jax.numpy as jnp
from jax.experimental import pallas as pl


def kernel(x, y, counts, known_mask):
    raise NotImplementedError("write your pallas kernel here")


# speedup vs baseline: 12.0383x; 12.0383x over previous
import functools

import jax
import jax.numpy as jnp
from jax import lax
from jax.experimental import pallas as pl
from jax.experimental.pallas import tpu as pltpu
from jax.experimental.pallas import tpu_sc as plsc


def _table_body(counts_ref, mask_ref, out_ref):
    c = counts_ref[...]
    nclass = c.shape[1]
    rowmax = jnp.max(c, axis=1, keepdims=True)
    io = lax.broadcasted_iota(jnp.int32, c.shape, 1)
    first = jnp.min(jnp.where(c >= rowmax, io, nclass), axis=1, keepdims=True)
    out_ref[...] = jnp.where(mask_ref[...] != 0, first, 1)


def _build_table(counts, mask_i32):
    V, C = counts.shape
    BV = 10000
    nb = V // BV
    return pl.pallas_call(
        _table_body,
        grid=(nb,),
        in_specs=[
            pl.BlockSpec((BV, C), lambda i: (i, 0)),
            pl.BlockSpec((BV, 1), lambda i: (i, 0)),
        ],
        out_specs=pl.BlockSpec((BV, 1), lambda i: (i, 0)),
        out_shape=jax.ShapeDtypeStruct((V, 1), jnp.int32),
    )(counts, mask_i32)


def _sc_gather(table_flat, idx_flat):
    (V,) = table_flat.shape
    (NT,) = idx_flat.shape
    info = plsc.get_sparse_core_info()
    NC, NS, L = info.num_cores, info.num_subcores, info.num_lanes
    NW = NC * NS
    bpw = NT // NW
    mesh = plsc.VectorSubcoreMesh(core_axis_name="c", subcore_axis_name="s")

    @functools.partial(
        pl.kernel,
        mesh=mesh,
        compiler_params=pltpu.CompilerParams(needs_layout_passes=False),
        out_type=jax.ShapeDtypeStruct((NT,), jnp.int32),
        scratch_types=[
            pltpu.VMEM((V,), jnp.int32),
            pltpu.VMEM((bpw,), jnp.int32),
            pltpu.VMEM((bpw,), jnp.int32),
        ],
    )
    def gather_k(table_hbm, idx_hbm, out_hbm, table_v, idx_v, out_v):
        wid = lax.axis_index("s") * NC + lax.axis_index("c")
        base = wid * bpw
        pltpu.sync_copy(table_hbm, table_v)
        pltpu.sync_copy(idx_hbm.at[pl.ds(base, bpw)], idx_v)

        def body(i, carry):
            ids = idx_v[pl.ds(i * L, L)]
            out_v[pl.ds(i * L, L)] = plsc.load_gather(table_v, [ids])
            return carry

        lax.fori_loop(0, bpw // L, body, 0)
        pltpu.sync_copy(out_v, out_hbm.at[pl.ds(base, bpw)])

    return gather_k(table_flat, idx_flat)


def _onehot_body(pred_ref, out_ref):
    p = pred_ref[...]
    io = lax.broadcasted_iota(jnp.int32, out_ref.shape, 1)
    out_ref[...] = (p == io).astype(jnp.float32)


def _build_onehot(preds_col, C):
    NT = preds_col.shape[0]
    BT = 4096
    nb = NT // BT
    return pl.pallas_call(
        _onehot_body,
        grid=(nb,),
        in_specs=[pl.BlockSpec((BT, 1), lambda i: (i, 0))],
        out_specs=pl.BlockSpec((BT, C), lambda i: (i, 0)),
        out_shape=jax.ShapeDtypeStruct((NT, C), jnp.float32),
    )(preds_col)


def kernel(x, y, counts, known_mask):
    n, t = x.shape
    V, C = counts.shape
    NT = n * t
    mask_i32 = known_mask.astype(jnp.int32).reshape(V, 1)
    table = _build_table(counts, mask_i32).reshape(V)
    xf = x.astype(jnp.int32).reshape(NT)
    preds = _sc_gather(table, xf)
    logits = _build_onehot(preds.reshape(NT, 1), C)
    y_hat = preds.reshape(n, t)
    return (logits.reshape(n, t, C), y, y_hat)


# speedup vs baseline: 57.7518x; 4.7973x over previous
import functools

import jax
import jax.numpy as jnp
from jax import lax
from jax.experimental import pallas as pl
from jax.experimental.pallas import tpu as pltpu
from jax.experimental.pallas import tpu_sc as plsc


def _table_body(countsT_ref, mask_ref, out_ref):
    c = countsT_ref[...]
    nclass = c.shape[0]
    colmax = jnp.max(c, axis=0, keepdims=True)
    io = lax.broadcasted_iota(jnp.int32, c.shape, 0)
    first = jnp.min(jnp.where(c >= colmax, io, nclass), axis=0)
    out_ref[...] = jnp.where(mask_ref[...] != 0, first, 1)


def _build_table(countsT, mask_i32):
    C, V = countsT.shape
    BV = 8192
    nb = pl.cdiv(V, BV)
    return pl.pallas_call(
        _table_body,
        grid=(nb,),
        in_specs=[
            pl.BlockSpec((C, BV), lambda i: (0, i)),
            pl.BlockSpec((BV,), lambda i: (i,)),
        ],
        out_specs=pl.BlockSpec((BV,), lambda i: (i,)),
        out_shape=jax.ShapeDtypeStruct((V,), jnp.int32),
    )(countsT, mask_i32)


def _sc_gather(table_flat, idx_flat):
    (V,) = table_flat.shape
    (NT,) = idx_flat.shape
    info = plsc.get_sparse_core_info()
    NC, NS, L = info.num_cores, info.num_subcores, info.num_lanes
    NW = NC * NS
    bpw = NT // NW
    mesh = plsc.VectorSubcoreMesh(core_axis_name="c", subcore_axis_name="s")

    @functools.partial(
        pl.kernel,
        mesh=mesh,
        compiler_params=pltpu.CompilerParams(needs_layout_passes=False),
        out_type=jax.ShapeDtypeStruct((NT,), jnp.int32),
        scratch_types=[
            pltpu.VMEM((V,), jnp.int32),
            pltpu.VMEM((bpw,), jnp.int32),
            pltpu.VMEM((bpw,), jnp.int32),
        ],
    )
    def gather_k(table_hbm, idx_hbm, out_hbm, table_v, idx_v, out_v):
        wid = lax.axis_index("s") * NC + lax.axis_index("c")
        base = wid * bpw
        pltpu.sync_copy(table_hbm, table_v)
        pltpu.sync_copy(idx_hbm.at[pl.ds(base, bpw)], idx_v)

        def body(i, carry):
            ids = idx_v[pl.ds(i * L, L)]
            out_v[pl.ds(i * L, L)] = plsc.load_gather(table_v, [ids])
            return carry

        lax.fori_loop(0, bpw // L, body, 0)
        pltpu.sync_copy(out_v, out_hbm.at[pl.ds(base, bpw)])

    return gather_k(table_flat, idx_flat)


def _onehot_body(pred_ref, out_ref):
    p = pred_ref[...]
    io = lax.broadcasted_iota(jnp.int32, out_ref.shape, 0)
    out_ref[...] = (p[None, :, :] == io).astype(jnp.float32)


def _build_onehot(predsT, C):
    T, N = predsT.shape
    BT = 8
    nb = T // BT
    return pl.pallas_call(
        _onehot_body,
        grid=(nb,),
        in_specs=[pl.BlockSpec((BT, N), lambda i: (i, 0))],
        out_specs=pl.BlockSpec((C, BT, N), lambda i: (0, i, 0)),
        out_shape=jax.ShapeDtypeStruct((C, T, N), jnp.float32),
    )(predsT)


def kernel(x, y, counts, known_mask):
    n, t = x.shape
    V, C = counts.shape
    NT = n * t
    countsT = counts.T
    mask_i32 = known_mask.astype(jnp.int32)
    table = _build_table(countsT, mask_i32)
    xT_flat = x.T.reshape(NT)
    preds = _sc_gather(table, xT_flat)
    predsT = preds.reshape(t, n)
    logitsT = _build_onehot(predsT, C)
    logits = jnp.transpose(logitsT, (2, 1, 0))
    y_hat = predsT.T
    return (logits, y, y_hat)
